# Initial kernel scaffold; baseline (speedup 1.0000x reference)
import jax
import jax.numpy as jnp
from jax.experimental import pallas as pl


def kernel(x, w1s, b1s, w2s, b2s, mixer, noise_controller):
    raise NotImplementedError("write your pallas kernel here")


# speedup vs baseline: 16.7247x; 16.7247x over previous
import functools

import jax
import jax.numpy as jnp
from jax.experimental import pallas as pl
from jax.experimental.pallas import tpu as pltpu

_B, _D_IN, _D_H, _D_OUT, _E, _K = 1024, 768, 64, 768, 16, 2
_BT = 256


def _moe_body(x_ref, mixer_ref, nc_ref, noise_ref, w1_ref, b1_ref, w2_ref,
              b2_ref, out_ref):
    x = x_ref[...]
    mixer_prod = jnp.dot(x, mixer_ref[...], preferred_element_type=jnp.float32)
    noise_prod = jnp.dot(x, nc_ref[...], preferred_element_type=jnp.float32)
    sp = jnp.maximum(noise_prod, 0.0) + jnp.log1p(jnp.exp(-jnp.abs(noise_prod)))
    h = mixer_prod + noise_ref[...] * sp
    iota = jax.lax.broadcasted_iota(jnp.int32, h.shape, 1)
    m1 = jnp.max(h, axis=1, keepdims=True)
    i1 = jnp.min(jnp.where(h == m1, iota, _E), axis=1, keepdims=True)
    hm = jnp.where(iota == i1, -jnp.inf, h)
    m2 = jnp.max(hm, axis=1, keepdims=True)
    i2 = jnp.min(jnp.where(hm == m2, iota, _E), axis=1, keepdims=True)
    e2 = jnp.exp(m2 - m1)
    denom = 1.0 + e2
    gates = (jnp.where(iota == i1, 1.0 / denom, 0.0)
             + jnp.where(iota == i2, e2 / denom, 0.0))
    ei = jax.lax.broadcasted_iota(jnp.int32, (_E, _E * _D_H), 0)
    ej = jax.lax.broadcasted_iota(jnp.int32, (_E, _E * _D_H), 1) // _D_H
    expand = (ei == ej).astype(jnp.float32)
    gate_wide = jnp.dot(gates, expand, preferred_element_type=jnp.float32)
    hid = jnp.dot(x, w1_ref[...], preferred_element_type=jnp.float32)
    hid = jnp.maximum(hid + b1_ref[...], 0.0) * gate_wide
    out = jnp.dot(hid, w2_ref[...], preferred_element_type=jnp.float32)
    out += jnp.dot(gates, b2_ref[...], preferred_element_type=jnp.float32)
    out_ref[...] = out


@jax.jit
def kernel(x, w1s, b1s, w2s, b2s, mixer, noise_controller):
    w1 = w1s.transpose(1, 0, 2).reshape(_D_IN, _E * _D_H)
    b1 = b1s.reshape(1, _E * _D_H)
    w2 = w2s.reshape(_E * _D_H, _D_OUT)
    noise = jax.random.normal(jax.random.key(42), (_B, _E), dtype=jnp.float32)
    grid = (_B // _BT,)
    return pl.pallas_call(
        _moe_body,
        grid=grid,
        in_specs=[
            pl.BlockSpec((_BT, _D_IN), lambda i: (i, 0)),
            pl.BlockSpec((_D_IN, _E), lambda i: (0, 0)),
            pl.BlockSpec((_D_IN, _E), lambda i: (0, 0)),
            pl.BlockSpec((_BT, _E), lambda i: (i, 0)),
            pl.BlockSpec((_D_IN, _E * _D_H), lambda i: (0, 0)),
            pl.BlockSpec((1, _E * _D_H), lambda i: (0, 0)),
            pl.BlockSpec((_E * _D_H, _D_OUT), lambda i: (0, 0)),
            pl.BlockSpec((_E, _D_OUT), lambda i: (0, 0)),
        ],
        out_specs=pl.BlockSpec((_BT, _D_OUT), lambda i: (i, 0)),
        out_shape=jax.ShapeDtypeStruct((_B, _D_OUT), jnp.float32),
        compiler_params=pltpu.CompilerParams(
            dimension_semantics=("arbitrary",)),
    )(x, mixer, noise_controller, noise, w1, b1, w2, b2s)
